# Initial kernel scaffold; baseline (speedup 1.0000x reference)
import jax
import jax.numpy as jnp
from jax.experimental import pallas as pl


def kernel(hidden_states, weight, e_score_correction_bias):
    raise NotImplementedError("write your pallas kernel here")


# speedup vs baseline: 14.1178x; 14.1178x over previous
import functools

import jax
import jax.numpy as jnp
from jax import lax
from jax.experimental import pallas as pl

N_GROUP = 8
TOPK_GROUP = 4
TOP_K = 8
ROUTED_SCALING_FACTOR = 2.5


def _gate_block(h_ref, w_ref, b_ref, o_ref):
    B = h_ref.shape[0]
    E = w_ref.shape[1]
    G = E // N_GROUP

    logits = jnp.dot(h_ref[...], w_ref[...], preferred_element_type=jnp.float32)
    lt = logits.T
    sig = jax.nn.sigmoid(lt)
    swb = sig + b_ref[...]

    neg = jnp.float32(-jnp.inf)

    iota_g = lax.broadcasted_iota(jnp.int32, (G, B), 0)
    group_scores = []
    for g in range(N_GROUP):
        c = swb[g * G:(g + 1) * G, :]
        m1 = jnp.max(c, axis=0, keepdims=True)
        eq = c == m1
        fi = jnp.min(jnp.where(eq, iota_g, G), axis=0, keepdims=True)
        m2 = jnp.max(jnp.where(iota_g == fi, neg, c), axis=0, keepdims=True)
        group_scores.append(m1 + m2)
    gsc = jnp.concatenate(group_scores, axis=0)

    iota_ng = lax.broadcasted_iota(jnp.int32, (N_GROUP, B), 0)
    gmask = jnp.zeros((N_GROUP, B), dtype=jnp.bool_)
    cur = gsc
    for _ in range(TOPK_GROUP):
        m = jnp.max(cur, axis=0, keepdims=True)
        eq = cur == m
        fi = jnp.min(jnp.where(eq, iota_ng, N_GROUP), axis=0, keepdims=True)
        sel = iota_ng == fi
        gmask = jnp.logical_or(gmask, sel)
        cur = jnp.where(sel, neg, cur)

    t = jnp.concatenate(
        [jnp.where(gmask[g:g + 1, :], swb[g * G:(g + 1) * G, :], 0.0)
         for g in range(N_GROUP)], axis=0)

    iota_e = lax.broadcasted_iota(jnp.int32, (E, B), 0)
    nm = jnp.zeros((E, B), dtype=jnp.bool_)
    for _ in range(TOP_K):
        m = jnp.max(t, axis=0, keepdims=True)
        eq = t == m
        fi = jnp.min(jnp.where(eq, iota_e, E), axis=0, keepdims=True)
        sel = iota_e == fi
        nm = jnp.logical_or(nm, sel)
        t = jnp.where(sel, neg, t)

    out = jnp.where(nm, sig, 0.0)
    s = jnp.sum(out, axis=0, keepdims=True) + 1e-20
    out = out * (ROUTED_SCALING_FACTOR / s)
    o_ref[...] = out.T


@functools.partial(jax.jit, static_argnames=("interpret",))
def kernel(hidden_states, weight, e_score_correction_bias, interpret=False):
    T, H = hidden_states.shape
    E = weight.shape[0]
    B = T
    for cand in (1024, 512, 256, 128, 64, 32, 16, 8):
        if T % cand == 0:
            B = cand
            break

    wT = weight.T
    bias = e_score_correction_bias.reshape(E, 1).astype(jnp.float32)

    return pl.pallas_call(
        _gate_block,
        grid=(T // B,),
        in_specs=[
            pl.BlockSpec((B, H), lambda i: (i, 0)),
            pl.BlockSpec((H, E), lambda i: (0, 0)),
            pl.BlockSpec((E, 1), lambda i: (0, 0)),
        ],
        out_specs=pl.BlockSpec((B, E), lambda i: (i, 0)),
        out_shape=jax.ShapeDtypeStruct((T, E), jnp.float32),
        interpret=interpret,
    )(hidden_states.astype(jnp.float32), wT, bias)


# speedup vs baseline: 15.3811x; 1.0895x over previous
import functools

import jax
import jax.numpy as jnp
from jax import lax
from jax.experimental import pallas as pl

N_GROUP = 8
TOPK_GROUP = 4
TOP_K = 8
ROUTED_SCALING_FACTOR = 2.5


def _gate_block(h_ref, w_ref, b_ref, o_ref):
    B = h_ref.shape[0]
    E = w_ref.shape[1]
    G = E // N_GROUP

    logits = jnp.dot(h_ref[...], w_ref[...], preferred_element_type=jnp.float32)
    lt = logits.T
    sig = jax.nn.sigmoid(lt)
    swb = sig + b_ref[...]

    neg = jnp.float32(-jnp.inf)

    iota_g = lax.broadcasted_iota(jnp.int32, (G, B), 0)
    group_scores = []
    for g in range(N_GROUP):
        c = swb[g * G:(g + 1) * G, :]
        m1 = jnp.max(c, axis=0, keepdims=True)
        eq = c == m1
        fi = jnp.min(jnp.where(eq, iota_g, G), axis=0, keepdims=True)
        m2 = jnp.max(jnp.where(iota_g == fi, neg, c), axis=0, keepdims=True)
        group_scores.append(m1 + m2)
    gsc = jnp.concatenate(group_scores, axis=0)

    iota_ng = lax.broadcasted_iota(jnp.int32, (N_GROUP, B), 0)
    gmask = jnp.zeros((N_GROUP, B), dtype=jnp.bool_)
    cur = gsc
    for _ in range(TOPK_GROUP):
        m = jnp.max(cur, axis=0, keepdims=True)
        eq = cur == m
        fi = jnp.min(jnp.where(eq, iota_ng, N_GROUP), axis=0, keepdims=True)
        sel = iota_ng == fi
        gmask = jnp.logical_or(gmask, sel)
        cur = jnp.where(sel, neg, cur)

    t = jnp.concatenate(
        [jnp.where(gmask[g:g + 1, :], swb[g * G:(g + 1) * G, :], 0.0)
         for g in range(N_GROUP)], axis=0)

    iota_e = lax.broadcasted_iota(jnp.int32, (E, B), 0)
    nm = jnp.zeros((E, B), dtype=jnp.bool_)
    for _ in range(TOP_K):
        m = jnp.max(t, axis=0, keepdims=True)
        eq = t == m
        fi = jnp.min(jnp.where(eq, iota_e, E), axis=0, keepdims=True)
        sel = iota_e == fi
        nm = jnp.logical_or(nm, sel)
        t = jnp.where(sel, neg, t)

    out = jnp.where(nm, sig, 0.0)
    s = jnp.sum(out, axis=0, keepdims=True) + 1e-20
    out = out * (ROUTED_SCALING_FACTOR / s)
    o_ref[...] = out.T


@functools.partial(jax.jit, static_argnames=("interpret",))
def kernel(hidden_states, weight, e_score_correction_bias, interpret=False):
    T, H = hidden_states.shape
    E = weight.shape[0]
    B = T
    for cand in (2048, 1024, 512, 256, 128, 64, 32, 16, 8):
        if T % cand == 0:
            B = cand
            break

    wT = weight.T
    bias = e_score_correction_bias.reshape(E, 1).astype(jnp.float32)

    return pl.pallas_call(
        _gate_block,
        grid=(T // B,),
        in_specs=[
            pl.BlockSpec((B, H), lambda i: (i, 0)),
            pl.BlockSpec((H, E), lambda i: (0, 0)),
            pl.BlockSpec((E, 1), lambda i: (0, 0)),
        ],
        out_specs=pl.BlockSpec((B, E), lambda i: (i, 0)),
        out_shape=jax.ShapeDtypeStruct((T, E), jnp.float32),
        interpret=interpret,
    )(hidden_states.astype(jnp.float32), wT, bias)


# speedup vs baseline: 15.7109x; 1.0214x over previous
import functools

import jax
import jax.numpy as jnp
from jax import lax
from jax.experimental import pallas as pl

N_GROUP = 8
TOPK_GROUP = 4
TOP_K = 8
ROUTED_SCALING_FACTOR = 2.5


def _gate_chunk(lt, b):
    E, C = lt.shape
    G = E // N_GROUP

    sig = jax.nn.sigmoid(lt)
    swb = sig + b

    neg = jnp.float32(-jnp.inf)

    iota_g = lax.broadcasted_iota(jnp.int32, (G, C), 0)
    group_scores = []
    for g in range(N_GROUP):
        c = swb[g * G:(g + 1) * G, :]
        m1 = jnp.max(c, axis=0, keepdims=True)
        eq = c == m1
        fi = jnp.min(jnp.where(eq, iota_g, G), axis=0, keepdims=True)
        m2 = jnp.max(jnp.where(iota_g == fi, neg, c), axis=0, keepdims=True)
        group_scores.append(m1 + m2)
    gsc = jnp.concatenate(group_scores, axis=0)

    iota_ng = lax.broadcasted_iota(jnp.int32, (N_GROUP, C), 0)
    gmask = jnp.zeros((N_GROUP, C), dtype=jnp.bool_)
    cur = gsc
    for _ in range(TOPK_GROUP):
        m = jnp.max(cur, axis=0, keepdims=True)
        eq = cur == m
        fi = jnp.min(jnp.where(eq, iota_ng, N_GROUP), axis=0, keepdims=True)
        sel = iota_ng == fi
        gmask = jnp.logical_or(gmask, sel)
        cur = jnp.where(sel, neg, cur)

    t = jnp.concatenate(
        [jnp.where(gmask[g:g + 1, :], swb[g * G:(g + 1) * G, :], 0.0)
         for g in range(N_GROUP)], axis=0)

    iota_e = lax.broadcasted_iota(jnp.int32, (E, C), 0)
    nm = jnp.zeros((E, C), dtype=jnp.bool_)
    for _ in range(TOP_K):
        m = jnp.max(t, axis=0, keepdims=True)
        eq = t == m
        fi = jnp.min(jnp.where(eq, iota_e, E), axis=0, keepdims=True)
        sel = iota_e == fi
        nm = jnp.logical_or(nm, sel)
        t = jnp.where(sel, neg, t)

    out = jnp.where(nm, sig, 0.0)
    s = jnp.sum(out, axis=0, keepdims=True) + 1e-20
    return out * (ROUTED_SCALING_FACTOR / s)


def _gate_block(h_ref, w_ref, b_ref, o_ref):
    B = h_ref.shape[0]

    logits = jnp.dot(h_ref[...], w_ref[...], preferred_element_type=jnp.float32)
    bias = b_ref[...]

    C = min(512, B)
    for c0 in range(0, B, C):
        lt = logits[c0:c0 + C, :].T
        o_ref[c0:c0 + C, :] = _gate_chunk(lt, bias).T


@functools.partial(jax.jit, static_argnames=("interpret",))
def kernel(hidden_states, weight, e_score_correction_bias, interpret=False):
    T, H = hidden_states.shape
    E = weight.shape[0]
    B = T
    for cand in (2048, 1024, 512, 256, 128, 64, 32, 16, 8):
        if T % cand == 0:
            B = cand
            break

    wT = weight.T
    bias = e_score_correction_bias.reshape(E, 1).astype(jnp.float32)

    return pl.pallas_call(
        _gate_block,
        grid=(T // B,),
        in_specs=[
            pl.BlockSpec((B, H), lambda i: (i, 0)),
            pl.BlockSpec((H, E), lambda i: (0, 0)),
            pl.BlockSpec((E, 1), lambda i: (0, 0)),
        ],
        out_specs=pl.BlockSpec((B, E), lambda i: (i, 0)),
        out_shape=jax.ShapeDtypeStruct((T, E), jnp.float32),
        interpret=interpret,
    )(hidden_states.astype(jnp.float32), wT, bias)


# speedup vs baseline: 15.7529x; 1.0027x over previous
import functools

import jax
import jax.numpy as jnp
from jax import lax
from jax.experimental import pallas as pl
from jax.experimental.pallas import tpu as pltpu

N_GROUP = 8
TOPK_GROUP = 4
TOP_K = 8
ROUTED_SCALING_FACTOR = 2.5


def _gate_chunk(lt, b):
    E, C = lt.shape
    G = E // N_GROUP

    sig = jax.nn.sigmoid(lt)
    swb = sig + b

    neg = jnp.float32(-jnp.inf)

    x = swb.reshape(N_GROUP, G, C)
    a1 = x
    a2 = jnp.full_like(x, neg)
    for k in (1, 2, 4):
        r1 = pltpu.roll(a1, k, axis=1)
        r2 = pltpu.roll(a2, k, axis=1)
        mn = jnp.minimum(a1, r1)
        a1 = jnp.maximum(a1, r1)
        a2 = jnp.maximum(jnp.maximum(a2, r2), mn)
    gsc = a1[:, 0, :] + a2[:, 0, :]

    iota_ng = lax.broadcasted_iota(jnp.int32, (N_GROUP, C), 0)
    gmask = jnp.zeros((N_GROUP, C), dtype=jnp.bool_)
    cur = gsc
    for _ in range(TOPK_GROUP):
        m = jnp.max(cur, axis=0, keepdims=True)
        eq = cur == m
        fi = jnp.min(jnp.where(eq, iota_ng, N_GROUP), axis=0, keepdims=True)
        sel = iota_ng == fi
        gmask = jnp.logical_or(gmask, sel)
        cur = jnp.where(sel, neg, cur)

    t = jnp.where(
        jnp.broadcast_to(gmask[:, None, :], (N_GROUP, G, C)), x, 0.0
    ).reshape(E, C)

    H = E // 2
    lo = t[:H, :]
    hi = t[H:, :]
    lowcur = lo >= hi
    tc = jnp.maximum(lo, hi)
    tmin = jnp.minimum(lo, hi)
    iota_h = lax.broadcasted_iota(jnp.int32, (H, C), 0)
    eidx = jnp.where(lowcur, iota_h, iota_h + H)
    nml = jnp.zeros((H, C), dtype=jnp.bool_)
    nmh = jnp.zeros((H, C), dtype=jnp.bool_)
    for _ in range(TOP_K):
        m = jnp.max(tc, axis=0, keepdims=True)
        eq = tc == m
        fi = jnp.min(jnp.where(eq, eidx, E), axis=0, keepdims=True)
        sel = eidx == fi
        nml = jnp.logical_or(nml, sel & lowcur)
        nmh = jnp.logical_or(nmh, sel & (~lowcur))
        lowcur = lowcur ^ sel
        tc_new = jnp.where(sel, tmin, tc)
        tmin = jnp.where(sel, neg, tmin)
        tc = tc_new
        eidx = jnp.where(sel, eidx ^ H, eidx)
    nm = jnp.concatenate([nml, nmh], axis=0)

    out = jnp.where(nm, sig, 0.0)
    s = jnp.sum(out, axis=0, keepdims=True) + 1e-20
    return out * (ROUTED_SCALING_FACTOR / s)


def _gate_block(h_ref, w_ref, b_ref, o_ref):
    B = h_ref.shape[0]

    logits = jnp.dot(h_ref[...], w_ref[...], preferred_element_type=jnp.float32)
    bias = b_ref[...]

    C = min(512, B)
    for c0 in range(0, B, C):
        lt = logits[c0:c0 + C, :].T
        o_ref[c0:c0 + C, :] = _gate_chunk(lt, bias).T


@functools.partial(jax.jit, static_argnames=("interpret",))
def kernel(hidden_states, weight, e_score_correction_bias, interpret=False):
    T, H = hidden_states.shape
    E = weight.shape[0]
    B = T
    for cand in (2048, 1024, 512, 256, 128, 64, 32, 16, 8):
        if T % cand == 0:
            B = cand
            break

    wT = weight.T
    bias = e_score_correction_bias.reshape(E, 1).astype(jnp.float32)

    return pl.pallas_call(
        _gate_block,
        grid=(T // B,),
        in_specs=[
            pl.BlockSpec((B, H), lambda i: (i, 0)),
            pl.BlockSpec((H, E), lambda i: (0, 0)),
            pl.BlockSpec((E, 1), lambda i: (0, 0)),
        ],
        out_specs=pl.BlockSpec((B, E), lambda i: (i, 0)),
        out_shape=jax.ShapeDtypeStruct((T, E), jnp.float32),
        interpret=interpret,
    )(hidden_states.astype(jnp.float32), wT, bias)
